# baseline (speedup 1.0000x reference)
import functools

import jax
import jax.numpy as jnp
from jax import lax
from jax.experimental import pallas as pl
from jax.experimental.pallas import tpu as pltpu
from jax.experimental.pallas import tpu_sc as plsc

N = 10000
E = 320000
D = 128
G = 64

NC = 2
NS = 16
NW = NC * NS
K = 80
CH_TOT = E // K
CH_PER_W = CH_TOT // NW
NB = 5
NB_A = 4
N_PAD = 10240
ROWS_PT = N_PAD // NS

_mesh = plsc.VectorSubcoreMesh(core_axis_name="c", subcore_axis_name="s")


@functools.partial(
    pl.kernel,
    out_type=jax.ShapeDtypeStruct((NC, N_PAD), jnp.float32),
    mesh=_mesh,
    scratch_types=[
        pltpu.VMEM((NB, 2, K), jnp.int32),
        pltpu.VMEM((1, K), jnp.float32),
        pltpu.VMEM_SHARED((N_PAD,), jnp.float32),
    ] + [pltpu.SemaphoreType.DMA] * (2 * NB),
)
def _deg_kernel(eidx_hbm, z1_hbm, deg_hbm, ij, ones, dacc, *sems):
    cid = lax.axis_index("c")
    sid = lax.axis_index("s")
    wid = sid * NC + cid
    isem = sems[:NB]
    ssem = sems[NB:]
    cbase = wid * CH_PER_W
    for i in range(K // 16):
        ones[0, pl.ds(i * 16, 16)] = jnp.full((16,), 1.0, jnp.float32)

    def idx_start(cc, j):
        pltpu.async_copy(eidx_hbm.at[cbase + cc], ij.at[j], isem[j])

    def idx_wait(cc, j):
        pltpu.make_async_copy(eidx_hbm.at[cbase + cc], ij.at[j],
                              isem[j]).wait()

    def sc_start(j):
        pltpu.async_copy(ones.at[0], dacc.at[ij.at[j, 1]], ssem[j], add=True)

    def sc_wait(j):
        pltpu.make_async_copy(ones.at[0], dacc.at[ij.at[j, 1]],
                              ssem[j]).wait()

    for j in range(NB - 1):
        idx_start(j, j)
    pltpu.sync_copy(z1_hbm, dacc.at[pl.ds(sid * ROWS_PT, ROWS_PT)])
    plsc.subcore_barrier()

    def block(i, _):
        for j in range(NB):
            cc = NB * i + j
            jm1 = (j - 1) % NB
            idx_wait(cc, j)
            sc_start(j)

            @pl.when(cc >= 1)
            def _():
                sc_wait(jm1)

            @pl.when(cc + NB - 1 < CH_PER_W)
            def _():
                idx_start(cc + NB - 1, jm1)
        return _

    lax.fori_loop(0, CH_PER_W // NB, block, None)
    sc_wait(NB - 1)
    plsc.subcore_barrier()
    pltpu.sync_copy(dacc.at[pl.ds(sid * ROWS_PT, ROWS_PT)],
                    deg_hbm.at[cid, pl.ds(sid * ROWS_PT, ROWS_PT)])


@functools.partial(
    pl.kernel,
    out_type=jax.ShapeDtypeStruct((NC, N_PAD, D), jnp.float32),
    mesh=_mesh,
    scratch_types=[
        pltpu.VMEM((NB_A, 2, K), jnp.int32),
        pltpu.VMEM((NB_A, K, D), jnp.float32),
        pltpu.VMEM_SHARED((N_PAD, D), jnp.float32),
    ] + [pltpu.SemaphoreType.DMA] * (3 * NB_A),
)
def _agg_kernel(g_hbm, eidx_hbm, z2_hbm, out_hbm, ij, rows, acc, *sems):
    cid = lax.axis_index("c")
    sid = lax.axis_index("s")
    wid = sid * NC + cid
    isem = sems[:NB_A]
    gsem = sems[NB_A:2 * NB_A]
    ssem = sems[2 * NB_A:]
    cbase = wid * CH_PER_W

    def idx_start(cc, j):
        pltpu.async_copy(eidx_hbm.at[cbase + cc], ij.at[j], isem[j])

    def idx_wait(cc, j):
        pltpu.make_async_copy(eidx_hbm.at[cbase + cc], ij.at[j],
                              isem[j]).wait()

    def g_start(j):
        pltpu.async_copy(g_hbm.at[ij.at[j, 0]], rows.at[j], gsem[j])

    def g_wait(j):
        pltpu.make_async_copy(g_hbm.at[ij.at[j, 0]], rows.at[j],
                              gsem[j]).wait()

    def sc_start(j):
        pltpu.async_copy(rows.at[j], acc.at[ij.at[j, 1]], ssem[j], add=True)

    def sc_wait(j):
        pltpu.make_async_copy(rows.at[j], acc.at[ij.at[j, 1]],
                              ssem[j]).wait()

    for j in range(NB_A - 1):
        idx_start(j, j)
    for j in range(NB_A - 2):
        idx_wait(j, j)
        g_start(j)
    pltpu.sync_copy(z2_hbm, acc.at[pl.ds(sid * ROWS_PT, ROWS_PT)])
    plsc.subcore_barrier()

    NFULL = CH_PER_W - 1

    def block(i, _):
        for j in range(NB_A):
            cc = NB_A * i + j
            jm1 = (j - 1) % NB_A
            jm2 = (j - 2) % NB_A
            g_wait(j)
            sc_start(j)

            @pl.when(cc >= 1)
            def _():
                sc_wait(jm1)

            @pl.when(cc + NB_A - 1 < CH_PER_W)
            def _():
                idx_start(cc + NB_A - 1, jm1)

            @pl.when(cc + NB_A - 2 < CH_PER_W)
            def _():
                idx_wait(cc + NB_A - 2, jm2)
                g_start(jm2)
        return _

    lax.fori_loop(0, NFULL // NB_A, block, None)
    g_wait(0)
    sc_start(0)
    sc_wait(NB_A - 1)
    sc_wait(0)
    plsc.subcore_barrier()
    pltpu.sync_copy(acc.at[pl.ds(sid * ROWS_PT, ROWS_PT)],
                    out_hbm.at[cid].at[pl.ds(sid * ROWS_PT, ROWS_PT)])


def _prep_a_body(x_ref, w1_ref, h1_ref):
    x = x_ref[...]
    mean = jnp.sum(x, axis=0, keepdims=True) / N
    msq = jnp.sum(x * x, axis=0, keepdims=True) / N
    var = jnp.maximum(msq - mean * mean, 0.0)
    std = jnp.sqrt(var)
    std = jnp.where(std == 0.0, 1.0, std)
    xs = (x - mean) / std
    h1_ref[...] = jnp.dot(xs, w1_ref[...], preferred_element_type=jnp.float32)


_prep_a = pl.pallas_call(
    _prep_a_body,
    out_shape=jax.ShapeDtypeStruct((N_PAD, D), jnp.float32),
)


def _prep_b_body(h1_ref, deg_ref, g1_ref, dinv_ref):
    deg = deg_ref[0] + deg_ref[1] + 1.0
    dinv = lax.rsqrt(deg)
    g1_ref[...] = h1_ref[...] * dinv
    dinv_ref[...] = dinv


_prep_b = pl.pallas_call(
    _prep_b_body,
    out_shape=[jax.ShapeDtypeStruct((N_PAD, D), jnp.float32),
               jax.ShapeDtypeStruct((N_PAD, 1), jnp.float32)],
)


def _mid_body(agg_ref, g1_ref, dinv_ref, b1_ref, w2_ref, g2_ref):
    t = agg_ref[0] + agg_ref[1] + g1_ref[...]
    x1 = jnp.maximum(t * dinv_ref[...] + b1_ref[...], 0.0)
    g2_ref[...] = jnp.dot(x1, w2_ref[...],
                          preferred_element_type=jnp.float32) * dinv_ref[...]


_mid = pl.pallas_call(
    _mid_body,
    out_shape=jax.ShapeDtypeStruct((N_PAD, D), jnp.float32),
)


def _fin_body(agg_ref, g2_ref, dinv_ref, b2_ref, batch_ref, wlin_ref,
              blin_ref, out_ref):
    x2 = (agg_ref[0] + agg_ref[1] + g2_ref[...]) * dinv_ref[...] + b2_ref[...]
    oh = (batch_ref[...] ==
          lax.broadcasted_iota(jnp.int32, (N_PAD, G), 1)).astype(jnp.float32)
    summed = lax.dot_general(oh, x2, (((0,), (0,)), ((), ())),
                             preferred_element_type=jnp.float32)
    counts = lax.dot_general(oh, jnp.ones((N_PAD, 1), jnp.float32),
                             (((0,), (0,)), ((), ())),
                             preferred_element_type=jnp.float32)
    pooled = summed / jnp.maximum(counts, 1.0)
    out_ref[...] = jnp.dot(pooled, wlin_ref[...],
                           preferred_element_type=jnp.float32) + blin_ref[...]


_fin = pl.pallas_call(
    _fin_body,
    out_shape=jax.ShapeDtypeStruct((G, D), jnp.float32),
)


def kernel(x, edge_index, batch, W1, b1, W2, b2, Wlin, blin):
    eidx = jnp.transpose(edge_index.reshape(2, CH_TOT, K), (1, 0, 2))
    xp = jnp.pad(x, ((0, N_PAD - N), (0, 0)))
    batchp = jnp.pad(batch, (0, N_PAD - N),
                     constant_values=G).reshape(N_PAD, 1)
    z1 = jnp.zeros((ROWS_PT,), jnp.float32)
    z2 = jnp.zeros((ROWS_PT, D), jnp.float32)

    deg = _deg_kernel(eidx, z1)
    h1 = _prep_a(xp, W1)
    g1, dinv = _prep_b(h1, deg.reshape(NC, N_PAD, 1))
    agg1 = _agg_kernel(g1, eidx, z2)
    g2 = _mid(agg1, g1, dinv, b1.reshape(1, D), W2)
    agg2 = _agg_kernel(g2, eidx, z2)
    out = _fin(agg2, g2, dinv, b2.reshape(1, D), batchp, Wlin,
               blin.reshape(1, D))
    return out


# speedup vs baseline: 1.0191x; 1.0191x over previous
import functools

import jax
import jax.numpy as jnp
from jax import lax
from jax.experimental import pallas as pl
from jax.experimental.pallas import tpu as pltpu
from jax.experimental.pallas import tpu_sc as plsc

N = 10000
E = 320000
D = 128
G = 64

NC = 2
NS = 16
NW = NC * NS
K = 80
CH_TOT = E // K
CH_PER_W = CH_TOT // NW
NB = 5
NB_A = 4
N_PAD = 10240
ROWS_PT = N_PAD // NS

_mesh = plsc.VectorSubcoreMesh(core_axis_name="c", subcore_axis_name="s")


@functools.partial(
    pl.kernel,
    out_type=jax.ShapeDtypeStruct((NC, N_PAD), jnp.float32),
    mesh=_mesh,
    scratch_types=[
        pltpu.VMEM((NB, K), jnp.int32),
        pltpu.VMEM((1, K), jnp.float32),
        pltpu.VMEM_SHARED((N_PAD,), jnp.float32),
    ] + [pltpu.SemaphoreType.DMA] * (2 * NB),
)
def _deg_kernel(eidx_hbm, z1_hbm, deg_hbm, ij, ones, dacc, *sems):
    cid = lax.axis_index("c")
    sid = lax.axis_index("s")
    wid = sid * NC + cid
    isem = sems[:NB]
    ssem = sems[NB:]
    cbase = wid * CH_PER_W
    for i in range(K // 16):
        ones[0, pl.ds(i * 16, 16)] = jnp.full((16,), 1.0, jnp.float32)

    def idx_start(cc, j):
        pltpu.async_copy(eidx_hbm.at[cbase + cc, 1], ij.at[j], isem[j])

    def idx_wait(cc, j):
        pltpu.make_async_copy(eidx_hbm.at[cbase + cc, 1], ij.at[j],
                              isem[j]).wait()

    def sc_start(j):
        pltpu.async_copy(ones.at[0], dacc.at[ij.at[j]], ssem[j], add=True)

    def sc_wait(j):
        pltpu.make_async_copy(ones.at[0], dacc.at[ij.at[j]],
                              ssem[j]).wait()

    for j in range(NB - 1):
        idx_start(j, j)
    pltpu.sync_copy(z1_hbm, dacc.at[pl.ds(sid * ROWS_PT, ROWS_PT)])
    plsc.subcore_barrier()

    def block(i, _):
        for j in range(NB):
            cc = NB * i + j
            jm1 = (j - 1) % NB
            idx_wait(cc, j)
            sc_start(j)

            @pl.when(cc >= 1)
            def _():
                sc_wait(jm1)

            @pl.when(cc + NB - 1 < CH_PER_W)
            def _():
                idx_start(cc + NB - 1, jm1)
        return _

    lax.fori_loop(0, CH_PER_W // NB, block, None)
    sc_wait(NB - 1)
    plsc.subcore_barrier()
    pltpu.sync_copy(dacc.at[pl.ds(sid * ROWS_PT, ROWS_PT)],
                    deg_hbm.at[cid, pl.ds(sid * ROWS_PT, ROWS_PT)])


@functools.partial(
    pl.kernel,
    out_type=jax.ShapeDtypeStruct((NC, N_PAD, D), jnp.float32),
    mesh=_mesh,
    scratch_types=[
        pltpu.VMEM((NB_A, 2, K), jnp.int32),
        pltpu.VMEM((NB_A, K, D), jnp.float32),
        pltpu.VMEM_SHARED((N_PAD, D), jnp.float32),
    ] + [pltpu.SemaphoreType.DMA] * (3 * NB_A),
)
def _agg_kernel(g_hbm, eidx_hbm, z2_hbm, out_hbm, ij, rows, acc, *sems):
    cid = lax.axis_index("c")
    sid = lax.axis_index("s")
    wid = sid * NC + cid
    isem = sems[:NB_A]
    gsem = sems[NB_A:2 * NB_A]
    ssem = sems[2 * NB_A:]
    cbase = wid * CH_PER_W

    def idx_start(cc, j):
        pltpu.async_copy(eidx_hbm.at[cbase + cc], ij.at[j], isem[j])

    def idx_wait(cc, j):
        pltpu.make_async_copy(eidx_hbm.at[cbase + cc], ij.at[j],
                              isem[j]).wait()

    def g_start(j):
        pltpu.async_copy(g_hbm.at[ij.at[j, 0]], rows.at[j], gsem[j])

    def g_wait(j):
        pltpu.make_async_copy(g_hbm.at[ij.at[j, 0]], rows.at[j],
                              gsem[j]).wait()

    def sc_start(j):
        pltpu.async_copy(rows.at[j], acc.at[ij.at[j, 1]], ssem[j], add=True)

    def sc_wait(j):
        pltpu.make_async_copy(rows.at[j], acc.at[ij.at[j, 1]],
                              ssem[j]).wait()

    for j in range(NB_A - 1):
        idx_start(j, j)
    for j in range(NB_A - 2):
        idx_wait(j, j)
        g_start(j)
    pltpu.sync_copy(z2_hbm, acc.at[pl.ds(sid * ROWS_PT, ROWS_PT)])
    plsc.subcore_barrier()

    NFULL = CH_PER_W - 1

    def block(i, _):
        for j in range(NB_A):
            cc = NB_A * i + j
            jm1 = (j - 1) % NB_A
            jm2 = (j - 2) % NB_A
            g_wait(j)
            sc_start(j)

            @pl.when(cc >= 1)
            def _():
                sc_wait(jm1)

            @pl.when(cc + NB_A - 1 < CH_PER_W)
            def _():
                idx_start(cc + NB_A - 1, jm1)

            @pl.when(cc + NB_A - 2 < CH_PER_W)
            def _():
                idx_wait(cc + NB_A - 2, jm2)
                g_start(jm2)
        return _

    lax.fori_loop(0, NFULL // NB_A, block, None)
    g_wait(0)
    sc_start(0)
    sc_wait(NB_A - 1)
    sc_wait(0)
    plsc.subcore_barrier()
    pltpu.sync_copy(acc.at[pl.ds(sid * ROWS_PT, ROWS_PT)],
                    out_hbm.at[cid].at[pl.ds(sid * ROWS_PT, ROWS_PT)])


def _prep_body(x_ref, deg_ref, w1_ref, g1_ref, dinv_ref):
    x = x_ref[...]
    mean = jnp.sum(x, axis=0, keepdims=True) / N
    msq = jnp.sum(x * x, axis=0, keepdims=True) / N
    var = jnp.maximum(msq - mean * mean, 0.0)
    std = jnp.sqrt(var)
    std = jnp.where(std == 0.0, 1.0, std)
    xs = (x - mean) / std
    xs = jnp.concatenate(
        [xs, jnp.zeros((N_PAD - N, D), jnp.float32)], axis=0)
    deg = deg_ref[0] + deg_ref[1] + 1.0
    dinv = lax.rsqrt(deg)
    g1_ref[...] = jnp.dot(xs, w1_ref[...],
                          preferred_element_type=jnp.float32) * dinv
    dinv_ref[...] = dinv


_prep = pl.pallas_call(
    _prep_body,
    out_shape=[jax.ShapeDtypeStruct((N_PAD, D), jnp.float32),
               jax.ShapeDtypeStruct((N_PAD, 1), jnp.float32)],
)


def _mid_body(agg_ref, g1_ref, dinv_ref, b1_ref, w2_ref, g2_ref):
    t = agg_ref[0] + agg_ref[1] + g1_ref[...]
    x1 = jnp.maximum(t * dinv_ref[...] + b1_ref[...], 0.0)
    g2_ref[...] = jnp.dot(x1, w2_ref[...],
                          preferred_element_type=jnp.float32) * dinv_ref[...]


_mid = pl.pallas_call(
    _mid_body,
    out_shape=jax.ShapeDtypeStruct((N_PAD, D), jnp.float32),
)


def _fin_body(agg_ref, g2_ref, dinv_ref, b2_ref, batch_ref, wlin_ref,
              blin_ref, out_ref):
    x2 = (agg_ref[0] + agg_ref[1] + g2_ref[...]) * dinv_ref[...] + b2_ref[...]
    oh = (batch_ref[...] ==
          lax.broadcasted_iota(jnp.int32, (N_PAD, G), 1)).astype(jnp.float32)
    summed = lax.dot_general(oh, x2, (((0,), (0,)), ((), ())),
                             preferred_element_type=jnp.float32)
    counts = lax.dot_general(oh, jnp.ones((N_PAD, 1), jnp.float32),
                             (((0,), (0,)), ((), ())),
                             preferred_element_type=jnp.float32)
    pooled = summed / jnp.maximum(counts, 1.0)
    out_ref[...] = jnp.dot(pooled, wlin_ref[...],
                           preferred_element_type=jnp.float32) + blin_ref[...]


_fin = pl.pallas_call(
    _fin_body,
    out_shape=jax.ShapeDtypeStruct((G, D), jnp.float32),
)


def kernel(x, edge_index, batch, W1, b1, W2, b2, Wlin, blin):
    eidx = jnp.transpose(edge_index.reshape(2, CH_TOT, K), (1, 0, 2))
    batchp = jnp.pad(batch, (0, N_PAD - N),
                     constant_values=G).reshape(N_PAD, 1)
    z1 = jnp.zeros((ROWS_PT,), jnp.float32)
    z2 = jnp.zeros((ROWS_PT, D), jnp.float32)

    deg = _deg_kernel(eidx, z1)
    g1, dinv = _prep(x, deg.reshape(NC, N_PAD, 1), W1)
    agg1 = _agg_kernel(g1, eidx, z2)
    g2 = _mid(agg1, g1, dinv, b1.reshape(1, D), W2)
    agg2 = _agg_kernel(g2, eidx, z2)
    out = _fin(agg2, g2, dinv, b2.reshape(1, D), batchp, Wlin,
               blin.reshape(1, D))
    return out


# speedup vs baseline: 1.0210x; 1.0018x over previous
import functools

import jax
import jax.numpy as jnp
from jax import lax
from jax.experimental import pallas as pl
from jax.experimental.pallas import tpu as pltpu
from jax.experimental.pallas import tpu_sc as plsc

N = 10000
E = 320000
D = 128
G = 64

NC = 2
NS = 16
NW = NC * NS
K = 80
CH_TOT = E // K
CH_PER_W = CH_TOT // NW
NB = 5
NB_A = 4
N_PAD = 10240
ROWS_PT = N_PAD // NS

_mesh = plsc.VectorSubcoreMesh(core_axis_name="c", subcore_axis_name="s")


@functools.partial(
    pl.kernel,
    out_type=jax.ShapeDtypeStruct((NC, N_PAD), jnp.float32),
    mesh=_mesh,
    scratch_types=[
        pltpu.VMEM((NB, K), jnp.int32),
        pltpu.VMEM((1, K), jnp.float32),
        pltpu.VMEM_SHARED((N_PAD,), jnp.float32),
    ] + [pltpu.SemaphoreType.DMA] * (2 * NB),
)
def _deg_kernel(eidx_hbm, z1_hbm, deg_hbm, ij, ones, dacc, *sems):
    cid = lax.axis_index("c")
    sid = lax.axis_index("s")
    wid = sid * NC + cid
    isem = sems[:NB]
    ssem = sems[NB:]
    cbase = wid * CH_PER_W
    for i in range(K // 16):
        ones[0, pl.ds(i * 16, 16)] = jnp.full((16,), 1.0, jnp.float32)

    def idx_start(cc, j):
        pltpu.async_copy(eidx_hbm.at[cbase + cc, 1], ij.at[j], isem[j])

    def idx_wait(cc, j):
        pltpu.make_async_copy(eidx_hbm.at[cbase + cc, 1], ij.at[j],
                              isem[j]).wait()

    def sc_start(j):
        pltpu.async_copy(ones.at[0], dacc.at[ij.at[j]], ssem[j], add=True)

    def sc_wait(j):
        pltpu.make_async_copy(ones.at[0], dacc.at[ij.at[j]],
                              ssem[j]).wait()

    for j in range(NB - 1):
        idx_start(j, j)
    pltpu.sync_copy(z1_hbm, dacc.at[pl.ds(sid * ROWS_PT, ROWS_PT)])
    plsc.subcore_barrier()

    def block(i, _):
        for j in range(NB):
            cc = NB * i + j
            jm1 = (j - 1) % NB
            idx_wait(cc, j)
            sc_start(j)

            @pl.when(cc >= 1)
            def _():
                sc_wait(jm1)

            @pl.when(cc + NB - 1 < CH_PER_W)
            def _():
                idx_start(cc + NB - 1, jm1)
        return _

    lax.fori_loop(0, CH_PER_W // NB, block, None)
    sc_wait(NB - 1)
    plsc.subcore_barrier()
    pltpu.sync_copy(dacc.at[pl.ds(sid * ROWS_PT, ROWS_PT)],
                    deg_hbm.at[cid, pl.ds(sid * ROWS_PT, ROWS_PT)])


@functools.partial(
    pl.kernel,
    out_type=jax.ShapeDtypeStruct((NC, N_PAD, D), jnp.float32),
    mesh=_mesh,
    scratch_types=[
        pltpu.VMEM((NB_A, 2, K), jnp.int32),
        pltpu.VMEM((NB_A, K, D), jnp.float32),
        pltpu.VMEM_SHARED((N_PAD, D), jnp.float32),
    ] + [pltpu.SemaphoreType.DMA] * (3 * NB_A),
)
def _agg_kernel(g_hbm, eidx_hbm, z2_hbm, out_hbm, ij, rows, acc, *sems):
    cid = lax.axis_index("c")
    sid = lax.axis_index("s")
    wid = sid * NC + cid
    isem = sems[:NB_A]
    gsem = sems[NB_A:2 * NB_A]
    ssem = sems[2 * NB_A:]
    cbase = wid * CH_PER_W

    def idx_start(cc, j):
        pltpu.async_copy(eidx_hbm.at[cbase + cc], ij.at[j], isem[j])

    def idx_wait(cc, j):
        pltpu.make_async_copy(eidx_hbm.at[cbase + cc], ij.at[j],
                              isem[j]).wait()

    def g_start(j):
        pltpu.async_copy(g_hbm.at[ij.at[j, 0]], rows.at[j], gsem[j])

    def g_wait(j):
        pltpu.make_async_copy(g_hbm.at[ij.at[j, 0]], rows.at[j],
                              gsem[j]).wait()

    def sc_start(j):
        pltpu.async_copy(rows.at[j], acc.at[ij.at[j, 1]], ssem[j], add=True)

    def sc_wait(j):
        pltpu.make_async_copy(rows.at[j], acc.at[ij.at[j, 1]],
                              ssem[j]).wait()

    for j in range(NB_A - 1):
        idx_start(j, j)
    for j in range(NB_A - 2):
        idx_wait(j, j)
        g_start(j)
    pltpu.sync_copy(z2_hbm, acc.at[pl.ds(sid * ROWS_PT, ROWS_PT)])
    plsc.subcore_barrier()

    NFULL = CH_PER_W - 1

    def block(i, _):
        for j in range(NB_A):
            cc = NB_A * i + j
            jm1 = (j - 1) % NB_A
            jm2 = (j - 2) % NB_A
            g_wait(j)
            sc_start(j)

            @pl.when(cc >= 1)
            def _():
                sc_wait(jm1)

            @pl.when(cc + NB_A - 1 < CH_PER_W)
            def _():
                idx_start(cc + NB_A - 1, jm1)

            @pl.when(cc + NB_A - 2 < CH_PER_W)
            def _():
                idx_wait(cc + NB_A - 2, jm2)
                g_start(jm2)
        return _

    lax.fori_loop(0, NFULL // NB_A, block, None)
    g_wait(0)
    sc_start(0)
    sc_wait(NB_A - 1)
    sc_wait(0)
    plsc.subcore_barrier()
    pltpu.sync_copy(acc.at[pl.ds(sid * ROWS_PT, ROWS_PT)],
                    out_hbm.at[cid].at[pl.ds(sid * ROWS_PT, ROWS_PT)])


def _prep_body(x_ref, deg_ref, w1_ref, g1_ref, dinv_ref):
    x = x_ref[...]
    mean = jnp.sum(x, axis=0, keepdims=True) / N
    msq = jnp.sum(x * x, axis=0, keepdims=True) / N
    var = jnp.maximum(msq - mean * mean, 0.0)
    std = jnp.sqrt(var)
    std = jnp.where(std == 0.0, 1.0, std)
    xs = (x - mean) / std
    xs = jnp.concatenate(
        [xs, jnp.zeros((N_PAD - N, D), jnp.float32)], axis=0)
    deg = deg_ref[0] + deg_ref[1] + 1.0
    dinv = lax.rsqrt(deg)
    g1_ref[...] = jnp.dot(xs.astype(jnp.bfloat16),
                          w1_ref[...].astype(jnp.bfloat16),
                          preferred_element_type=jnp.float32) * dinv
    dinv_ref[...] = dinv


_prep = pl.pallas_call(
    _prep_body,
    out_shape=[jax.ShapeDtypeStruct((N_PAD, D), jnp.float32),
               jax.ShapeDtypeStruct((N_PAD, 1), jnp.float32)],
)


def _mid_body(agg_ref, g1_ref, dinv_ref, b1_ref, w2_ref, g2_ref):
    t = agg_ref[0] + agg_ref[1] + g1_ref[...]
    x1 = jnp.maximum(t * dinv_ref[...] + b1_ref[...], 0.0)
    g2_ref[...] = jnp.dot(x1.astype(jnp.bfloat16),
                          w2_ref[...].astype(jnp.bfloat16),
                          preferred_element_type=jnp.float32) * dinv_ref[...]


_mid = pl.pallas_call(
    _mid_body,
    out_shape=jax.ShapeDtypeStruct((N_PAD, D), jnp.float32),
)


def _fin_body(agg_ref, g2_ref, dinv_ref, b2_ref, batch_ref, wlin_ref,
              blin_ref, out_ref):
    x2 = (agg_ref[0] + agg_ref[1] + g2_ref[...]) * dinv_ref[...] + b2_ref[...]
    oh = (batch_ref[...] ==
          lax.broadcasted_iota(jnp.int32, (N_PAD, G), 1)).astype(jnp.bfloat16)
    summed = lax.dot_general(oh, x2.astype(jnp.bfloat16),
                             (((0,), (0,)), ((), ())),
                             preferred_element_type=jnp.float32)
    counts = lax.dot_general(oh, jnp.ones((N_PAD, 1), jnp.bfloat16),
                             (((0,), (0,)), ((), ())),
                             preferred_element_type=jnp.float32)
    pooled = summed / jnp.maximum(counts, 1.0)
    out_ref[...] = jnp.dot(pooled, wlin_ref[...],
                           preferred_element_type=jnp.float32) + blin_ref[...]


_fin = pl.pallas_call(
    _fin_body,
    out_shape=jax.ShapeDtypeStruct((G, D), jnp.float32),
)


def kernel(x, edge_index, batch, W1, b1, W2, b2, Wlin, blin):
    eidx = jnp.transpose(edge_index.reshape(2, CH_TOT, K), (1, 0, 2))
    batchp = jnp.pad(batch, (0, N_PAD - N),
                     constant_values=G).reshape(N_PAD, 1)
    z1 = jnp.zeros((ROWS_PT,), jnp.float32)
    z2 = jnp.zeros((ROWS_PT, D), jnp.float32)

    deg = _deg_kernel(eidx, z1)
    g1, dinv = _prep(x, deg.reshape(NC, N_PAD, 1), W1)
    agg1 = _agg_kernel(g1, eidx, z2)
    g2 = _mid(agg1, g1, dinv, b1.reshape(1, D), W2)
    agg2 = _agg_kernel(g2, eidx, z2)
    out = _fin(agg2, g2, dinv, b2.reshape(1, D), batchp, Wlin,
               blin.reshape(1, D))
    return out
